# Initial kernel scaffold; baseline (speedup 1.0000x reference)
import jax
import jax.numpy as jnp
from jax.experimental import pallas as pl


def kernel(x, cond_sizes, weight):
    raise NotImplementedError("write your pallas kernel here")


# speedup vs baseline: 63.8908x; 63.8908x over previous
import functools

import jax
import jax.numpy as jnp
from jax.experimental import pallas as pl
from jax.experimental.pallas import tpu as pltpu

_N_PIECES = 20


def _fspool_block(n_pieces, cb, s, sizes_ref, x_ref, w_ref, pooled_ref, perm_ref):
    b = pl.program_id(0)
    size = sizes_ref[b]

    j_i32 = jax.lax.broadcasted_iota(jnp.int32, (1, s), 1)
    j_f = j_i32.astype(jnp.float32)
    denom = jnp.maximum(size - 1, 1).astype(jnp.float32)
    st = jnp.where(size == 1, j_f + 1.0 / denom, j_f / denom)
    valid = st <= 1.0
    t = n_pieces * jnp.minimum(st, 1.0)

    v = x_ref[0] + jnp.where(valid, 0.0, -99999.0)
    idx = jax.lax.broadcasted_iota(jnp.int32, (cb, s), 1)

    mask_cache = {}

    def _low_mask(d):
        if d not in mask_cache:
            mask_cache[d] = (j_i32 & d) == 0
        return mask_cache[d]

    k = 2
    while k <= s:
        mk = _low_mask(k)
        j = k // 2
        while j >= 1:
            mj = _low_mask(j)
            take_max = mk == mj
            pv = jnp.where(mj, jnp.roll(v, -j, axis=1), jnp.roll(v, j, axis=1))
            pi = jnp.where(mj, jnp.roll(idx, -j, axis=1), jnp.roll(idx, j, axis=1))
            self_greater = (v > pv) | ((v == pv) & (idx > pi))
            keep = take_max == self_greater
            v = jnp.where(keep, v, pv)
            idx = jnp.where(keep, idx, pi)
            j //= 2
        k *= 2

    perm_ref[0] = idx

    svm = v * valid.astype(jnp.float32)
    wt = w_ref
    w_full = jnp.zeros((cb, s), jnp.float32)
    for p in range(n_pieces + 1):
        hat = jnp.maximum(1.0 - jnp.abs(t - float(p)), 0.0)
        w_full = w_full + wt[:, p : p + 1] * hat
    pooled_ref[0] = jnp.sum(svm * w_full, axis=1, keepdims=True)


def _fspool(x, cond_sizes, weight, interpret=False):
    bsz, c, s = x.shape
    n_pieces = weight.shape[1] - 1
    cb = min(32, c)

    grid_spec = pltpu.PrefetchScalarGridSpec(
        num_scalar_prefetch=1,
        grid=(bsz, c // cb),
        in_specs=[
            pl.BlockSpec((1, cb, s), lambda b, ci, sref: (b, ci, 0)),
            pl.BlockSpec((cb, n_pieces + 1), lambda b, ci, sref: (ci, 0)),
        ],
        out_specs=[
            pl.BlockSpec((1, cb, 1), lambda b, ci, sref: (b, ci, 0)),
            pl.BlockSpec((1, cb, s), lambda b, ci, sref: (b, ci, 0)),
        ],
    )
    pooled3, perm = pl.pallas_call(
        functools.partial(_fspool_block, n_pieces, cb, s),
        grid_spec=grid_spec,
        out_shape=[
            jax.ShapeDtypeStruct((bsz, c, 1), jnp.float32),
            jax.ShapeDtypeStruct((bsz, c, s), jnp.int32),
        ],
        compiler_params=pltpu.CompilerParams(
            dimension_semantics=("parallel", "parallel"),
        ),
        interpret=interpret,
    )(cond_sizes.astype(jnp.int32), x, weight)
    return pooled3.reshape(bsz, c), perm


def kernel(x, cond_sizes, weight):
    return _fspool(x, cond_sizes, weight)
